# baseline (speedup 1.0000x reference)
import functools

import jax
import jax.numpy as jnp
from jax import lax
from jax.experimental import pallas as pl
from jax.experimental.pallas import tpu as pltpu
from jax.experimental.pallas import tpu_sc as plsc

_NC = 2
_NS = 16
_NW = _NC * _NS
_B = 128

_DNUMS = lax.GatherDimensionNumbers(
    offset_dims=(), collapsed_slice_dims=(0,), start_index_map=(0,)
)


def _bcast_lane(v, k):
    kv = jnp.full((16, 1), k, jnp.int32)
    return lax.gather(
        v, kv, _DNUMS, (1,), mode=lax.GatherScatterMode.PROMISE_IN_BOUNDS
    )


def _hermite_weights(t):
    t2 = t * t
    t3 = t2 * t
    w0 = -0.5 * t3 + t2 - 0.5 * t
    w1 = 1.5 * t3 - 2.5 * t2 + 1.0
    w2 = -1.5 * t3 + 2.0 * t2 + 0.5 * t
    w3 = 0.5 * t3 - 0.5 * t2
    return w0, w1, w2, w3


def _prep_block(x_ref, y_ref, idx_ref, w_ref, *, Hc, Wc, bR):
    x = jnp.clip(x_ref[...], 1.0, float(Hc) - 3.0)[:, None, :]
    y = jnp.clip(y_ref[...], 1.0, float(Wc) - 3.0)[:, None, :]
    bxf = jnp.floor(x)
    byf = jnp.floor(y)
    tx = x - bxf
    ty = y - byf
    bx = bxf.astype(jnp.int32)
    by = byf.astype(jnp.int32)

    wx0, wx1, wx2, wx3 = _hermite_weights(tx)
    wy0, wy1, wy2, wy3 = _hermite_weights(ty)

    k = lax.broadcasted_iota(jnp.int32, (bR, 16, 128), 1)
    di = k // 4
    dj = k % 4

    wx = jnp.where(di == 0, wx0, jnp.where(di == 1, wx1, jnp.where(di == 2, wx2, wx3)))
    wy = jnp.where(dj == 0, wy0, jnp.where(dj == 1, wy1, jnp.where(dj == 2, wy2, wy3)))
    w_ref[...] = wx * wy
    idx_ref[...] = (bx + di - 1) * Wc + (by + dj - 1)


@functools.cache
def _build(N, H, W, C):
    R = N // _B
    bR = 256

    prep = pl.pallas_call(
        functools.partial(_prep_block, Hc=H, Wc=W, bR=bR),
        grid=(R // bR,),
        in_specs=[
            pl.BlockSpec((bR, _B), lambda i: (i, 0)),
            pl.BlockSpec((bR, _B), lambda i: (i, 0)),
        ],
        out_specs=[
            pl.BlockSpec((bR, 16, _B), lambda i: (i, 0, 0)),
            pl.BlockSpec((bR, 16, _B), lambda i: (i, 0, 0)),
        ],
        out_shape=[
            jax.ShapeDtypeStruct((R, 16, _B), jnp.int32),
            jax.ShapeDtypeStruct((R, 16, _B), jnp.float32),
        ],
    )

    nchunks = R // _NW
    mesh = plsc.VectorSubcoreMesh(core_axis_name="c", subcore_axis_name="s")
    hper = H // _NW

    @functools.partial(
        pl.kernel,
        mesh=mesh,
        out_type=jax.ShapeDtypeStruct((H * W * C // 128, 128), jnp.float32),
        scratch_types=[
            pltpu.VMEM((2, 16, 8, 129), jnp.float32),
            pltpu.VMEM((2, 128, 128), jnp.float32),
            pltpu.SemaphoreType.DMA,
            pltpu.SemaphoreType.DMA,
            pltpu.SemaphoreType.DMA,
            pltpu.SemaphoreType.DMA,
        ],
        compiler_params=pltpu.CompilerParams(
            use_tc_tiling_on_sc=False, needs_layout_passes=False
        ),
    )
    def sc_relayout(v_hbm, t_hbm, src_v, out_v, lsem0, lsem1, osem0, osem1):
        lsem = (lsem0, lsem1)
        osem = (osem0, osem1)
        iot16 = lax.iota(jnp.int32, 16)
        wid = lax.axis_index("s") * _NC + lax.axis_index("c")
        steps = hper * 2

        def src_copies(s, b):
            h = wid * hper + s // 2
            half = s % 2
            return [
                pltpu.make_async_copy(
                    v_hbm.at[h * 2 + ci // 8, pl.ds(half * 8, 8), ci % 8],
                    src_v.at[b, ci, :, pl.ds(0, _B)],
                    lsem[b],
                )
                for ci in range(16)
            ]

        def out_copy(s, b):
            h = wid * hper + s // 2
            half = s % 2
            r0 = pl.multiple_of(h * (W * C // 128) + half * 128, 128)
            return pltpu.make_async_copy(
                out_v.at[b], t_hbm.at[pl.ds(r0, 128)], osem[b]
            )

        for cp in src_copies(0, 0):
            cp.start()
        for cp in src_copies(1, 1):
            cp.start()

        @pl.loop(0, steps, step=2)
        def _pair(si):
            for b in (0, 1):
                s = si + b
                for cp in src_copies(s, b):
                    cp.wait()

                @pl.when(s >= 2)
                def _():
                    out_copy(s - 2, b).wait()

                @plsc.parallel_loop(0, 128, unroll=2)
                def _(r):
                    w0 = r * 8
                    wbv = jnp.full((16,), w0 // _B, jnp.int32)
                    for j in range(8):
                        wlv = jnp.full((16,), w0 % _B + j, jnp.int32)
                        vals = plsc.load_gather(src_v.at[b], [iot16, wbv, wlv])
                        out_v[b, r, pl.ds(j * 16, 16)] = vals

                out_copy(s, b).start()

                @pl.when(s + 2 < steps)
                def _():
                    for cp in src_copies(s + 2, b):
                        cp.start()

        out_copy(steps - 2, 0).wait()
        out_copy(steps - 1, 1).wait()

    @functools.partial(
        pl.kernel,
        mesh=mesh,
        out_type=jax.ShapeDtypeStruct((C // 8, R, 8, _B), jnp.float32),
        scratch_types=[
            pltpu.VMEM((2, 16, _B), jnp.int32),
            pltpu.VMEM((2, 16, _B), jnp.float32),
            pltpu.VMEM((2, 16 * _B, C), jnp.float32),
            pltpu.VMEM((2, C, _B), jnp.float32),
            pltpu.SemaphoreType.DMA,
            pltpu.SemaphoreType.DMA,
            pltpu.SemaphoreType.DMA,
            pltpu.SemaphoreType.DMA,
            pltpu.SemaphoreType.DMA,
            pltpu.SemaphoreType.DMA,
            pltpu.SemaphoreType.DMA,
            pltpu.SemaphoreType.DMA,
        ],
        compiler_params=pltpu.CompilerParams(
            use_tc_tiling_on_sc=False, needs_layout_passes=False
        ),
    )
    def sc_main(
        table_ref,
        idx_hbm,
        w_hbm,
        out_hbm,
        idx_v,
        w_v,
        rows_v,
        out_v,
        isem0,
        isem1,
        wsem0,
        wsem1,
        gsem0,
        gsem1,
        osem0,
        osem1,
    ):
        isem = (isem0, isem1)
        wsem = (wsem0, wsem1)
        gsem = (gsem0, gsem1)
        osem = (osem0, osem1)

        iot16 = lax.iota(jnp.int32, 16)
        wid = lax.axis_index("s") * _NC + lax.axis_index("c")
        base_g = wid * nchunks

        def idx_copy(g, b):
            src = idx_hbm.at[pl.ds(pl.multiple_of(g * 16, 16), 16)]
            return pltpu.make_async_copy(src, idx_v.at[b], isem[b])

        def w_copy(g, b):
            src = w_hbm.at[pl.ds(pl.multiple_of(g * 16, 16), 16)]
            return pltpu.make_async_copy(src, w_v.at[b], wsem[b])

        def gather_copy(j, b):
            return pltpu.make_async_copy(
                table_ref.at[idx_v.at[b, j]],
                rows_v.at[b, pl.ds(j * _B, _B)],
                gsem[b],
            )

        def out_copy(g, b):
            return (
                pltpu.make_async_copy(
                    out_v.at[b, pl.ds(0, 8)], out_hbm.at[0, g], osem[b]
                ),
                pltpu.make_async_copy(
                    out_v.at[b, pl.ds(8, 8)], out_hbm.at[1, g], osem[b]
                ),
            )

        def combine(b):
            @plsc.parallel_loop(0, _B, unroll=2)
            def _(p):
                pcol = jnp.full((16,), p, jnp.int32)
                wv = plsc.load_gather(w_v.at[b], [iot16, pcol])
                t = [
                    rows_v[b, k * _B + p] * _bcast_lane(wv, k) for k in range(16)
                ]
                while len(t) > 1:
                    t = [t[i] + t[i + 1] for i in range(0, len(t), 2)]
                plsc.store_scatter(out_v.at[b], [iot16, pcol], t[0])

        idx_copy(base_g, 0).start()
        w_copy(base_g, 0).start()
        idx_copy(base_g + 1, 1).start()
        w_copy(base_g + 1, 1).start()
        idx_copy(base_g, 0).wait()
        for j in range(16):
            gather_copy(j, 0).start()

        @pl.loop(0, nchunks, step=2)
        def _pair(ci):
            for b in (0, 1):
                c = ci + b
                g = base_g + c
                nb = 1 - b

                @pl.when(c + 1 < nchunks)
                def _():
                    idx_copy(g + 1, nb).wait()
                    for j in range(16):
                        gather_copy(j, nb).start()

                for j in range(16):
                    gather_copy(j, b).wait()

                @pl.when(c + 2 < nchunks)
                def _():
                    idx_copy(g + 2, b).start()

                @pl.when(c >= 2)
                def _():
                    for cp in out_copy(g - 2, b):
                        cp.wait()

                w_copy(g, b).wait()
                combine(b)
                for cp in out_copy(g, b):
                    cp.start()

                @pl.when(c + 2 < nchunks)
                def _():
                    w_copy(g + 2, b).start()

        for cp in out_copy(base_g + nchunks - 2, 0):
            cp.wait()
        for cp in out_copy(base_g + nchunks - 1, 1):
            cp.wait()

    return prep, sc_relayout, sc_main


def kernel(coords, visible):
    H, W, C = visible.shape
    N = coords.shape[0]
    R = N // _B
    prep, sc_relayout, sc_main = _build(N, H, W, C)
    xs = coords[:, 0].reshape(R, _B)
    ys = coords[:, 1].reshape(R, _B)
    idx3, w3 = prep(xs, ys)
    v_tiled = (
        visible.reshape(H, W // 128, 128, C // 8, 8)
        .transpose(0, 3, 1, 4, 2)
        .reshape(H * (C // 8), W // 128, 8, 128)
    )
    table = sc_relayout(v_tiled).reshape(H * W, C)
    out = sc_main(table, idx3.reshape(R * 16, _B), w3.reshape(R * 16, _B))
    return out.transpose(1, 3, 0, 2).reshape(N, C)


# speedup vs baseline: 1.0532x; 1.0532x over previous
import functools

import jax
import jax.numpy as jnp
from jax import lax
from jax.experimental import pallas as pl
from jax.experimental.pallas import tpu as pltpu
from jax.experimental.pallas import tpu_sc as plsc

_NC = 2
_NS = 16
_NW = _NC * _NS
_B = 128

_DNUMS = lax.GatherDimensionNumbers(
    offset_dims=(), collapsed_slice_dims=(0,), start_index_map=(0,)
)


def _bcast_lane(v, k):
    kv = jnp.full((16, 1), k, jnp.int32)
    return lax.gather(
        v, kv, _DNUMS, (1,), mode=lax.GatherScatterMode.PROMISE_IN_BOUNDS
    )


def _hermite_weights(t):
    t2 = t * t
    t3 = t2 * t
    w0 = -0.5 * t3 + t2 - 0.5 * t
    w1 = 1.5 * t3 - 2.5 * t2 + 1.0
    w2 = -1.5 * t3 + 2.0 * t2 + 0.5 * t
    w3 = 0.5 * t3 - 0.5 * t2
    return w0, w1, w2, w3


def _prep_block(x_ref, y_ref, idx_ref, w_ref, *, Hc, Wc, bR):
    x = jnp.clip(x_ref[...], 1.0, float(Hc) - 3.0)[:, None, :]
    y = jnp.clip(y_ref[...], 1.0, float(Wc) - 3.0)[:, None, :]
    bxf = jnp.floor(x)
    byf = jnp.floor(y)
    tx = x - bxf
    ty = y - byf
    bx = bxf.astype(jnp.int32)
    by = byf.astype(jnp.int32)

    wx0, wx1, wx2, wx3 = _hermite_weights(tx)
    wy0, wy1, wy2, wy3 = _hermite_weights(ty)

    k = lax.broadcasted_iota(jnp.int32, (bR, 16, 128), 1)
    di = k // 4
    dj = k % 4

    wx = jnp.where(di == 0, wx0, jnp.where(di == 1, wx1, jnp.where(di == 2, wx2, wx3)))
    wy = jnp.where(dj == 0, wy0, jnp.where(dj == 1, wy1, jnp.where(dj == 2, wy2, wy3)))
    w_ref[...] = wx * wy
    idx_ref[...] = (bx + di - 1) * Wc + (by + dj - 1)


@functools.cache
def _build(N, H, W, C):
    R = N // _B
    bR = 256

    prep = pl.pallas_call(
        functools.partial(_prep_block, Hc=H, Wc=W, bR=bR),
        grid=(R // bR,),
        in_specs=[
            pl.BlockSpec((bR, _B), lambda i: (i, 0)),
            pl.BlockSpec((bR, _B), lambda i: (i, 0)),
        ],
        out_specs=[
            pl.BlockSpec((bR, 16, _B), lambda i: (i, 0, 0)),
            pl.BlockSpec((bR, 16, _B), lambda i: (i, 0, 0)),
        ],
        out_shape=[
            jax.ShapeDtypeStruct((R, 16, _B), jnp.int32),
            jax.ShapeDtypeStruct((R, 16, _B), jnp.float32),
        ],
    )

    nchunks = R // _NW
    mesh = plsc.VectorSubcoreMesh(core_axis_name="c", subcore_axis_name="s")
    hper = H // _NW

    @functools.partial(
        pl.kernel,
        mesh=mesh,
        out_type=jax.ShapeDtypeStruct((H * W * C // 128, 128), jnp.float32),
        scratch_types=[
            pltpu.VMEM((2, 16, 8, 128), jnp.float32),
            pltpu.VMEM((2, 128, 128), jnp.float32),
            pltpu.SemaphoreType.DMA,
            pltpu.SemaphoreType.DMA,
            pltpu.SemaphoreType.DMA,
            pltpu.SemaphoreType.DMA,
        ],
        compiler_params=pltpu.CompilerParams(
            use_tc_tiling_on_sc=False, needs_layout_passes=False
        ),
    )
    def sc_relayout(v_hbm, t_hbm, src_v, out_v, lsem0, lsem1, osem0, osem1):
        lsem = (lsem0, lsem1)
        osem = (osem0, osem1)
        iot16 = lax.iota(jnp.int32, 16)
        c8v8 = (iot16 // 8) * 8
        c8r = iot16 % 8
        wid = lax.axis_index("s") * _NC + lax.axis_index("c")
        steps = hper * 2

        def src_copies(s, b):
            h = wid * hper + s // 2
            half = s % 2
            return [
                pltpu.make_async_copy(
                    v_hbm.at[pl.ds((h * 2 + c8i) * 16 + half * 8, 8)],
                    src_v.at[b, pl.ds(c8i * 8, 8)],
                    lsem[b],
                )
                for c8i in (0, 1)
            ]

        def out_copy(s, b):
            h = wid * hper + s // 2
            half = s % 2
            r0 = pl.multiple_of(h * (W * C // 128) + half * 128, 128)
            return pltpu.make_async_copy(
                out_v.at[b], t_hbm.at[pl.ds(r0, 128)], osem[b]
            )

        for cp in src_copies(0, 0):
            cp.start()
        for cp in src_copies(1, 1):
            cp.start()

        @pl.loop(0, steps, step=2)
        def _pair(si):
            for b in (0, 1):
                s = si + b
                for cp in src_copies(s, b):
                    cp.wait()

                @pl.when(s >= 2)
                def _():
                    out_copy(s - 2, b).wait()

                @plsc.parallel_loop(0, 128, unroll=2)
                def _(r):
                    w0 = r * 8
                    idx0 = c8v8 + w0 // _B
                    for j in range(8):
                        wlv = jnp.full((16,), w0 % _B + j, jnp.int32)
                        vals = plsc.load_gather(src_v.at[b], [idx0, c8r, wlv])
                        out_v[b, r, pl.ds(j * 16, 16)] = vals

                out_copy(s, b).start()

                @pl.when(s + 2 < steps)
                def _():
                    for cp in src_copies(s + 2, b):
                        cp.start()

        out_copy(steps - 2, 0).wait()
        out_copy(steps - 1, 1).wait()

    @functools.partial(
        pl.kernel,
        mesh=mesh,
        out_type=jax.ShapeDtypeStruct((C // 8, R, 8, _B), jnp.float32),
        scratch_types=[
            pltpu.VMEM((2, 16, _B), jnp.int32),
            pltpu.VMEM((2, 16, _B), jnp.float32),
            pltpu.VMEM((2, 16 * _B, C), jnp.float32),
            pltpu.VMEM((2, C, _B), jnp.float32),
            pltpu.SemaphoreType.DMA,
            pltpu.SemaphoreType.DMA,
            pltpu.SemaphoreType.DMA,
            pltpu.SemaphoreType.DMA,
            pltpu.SemaphoreType.DMA,
            pltpu.SemaphoreType.DMA,
            pltpu.SemaphoreType.DMA,
            pltpu.SemaphoreType.DMA,
        ],
        compiler_params=pltpu.CompilerParams(
            use_tc_tiling_on_sc=False, needs_layout_passes=False
        ),
    )
    def sc_main(
        table_ref,
        idx_hbm,
        w_hbm,
        out_hbm,
        idx_v,
        w_v,
        rows_v,
        out_v,
        isem0,
        isem1,
        wsem0,
        wsem1,
        gsem0,
        gsem1,
        osem0,
        osem1,
    ):
        isem = (isem0, isem1)
        wsem = (wsem0, wsem1)
        gsem = (gsem0, gsem1)
        osem = (osem0, osem1)

        iot16 = lax.iota(jnp.int32, 16)
        wid = lax.axis_index("s") * _NC + lax.axis_index("c")
        base_g = wid * nchunks

        def idx_copy(g, b):
            src = idx_hbm.at[pl.ds(pl.multiple_of(g * 16, 16), 16)]
            return pltpu.make_async_copy(src, idx_v.at[b], isem[b])

        def w_copy(g, b):
            src = w_hbm.at[pl.ds(pl.multiple_of(g * 16, 16), 16)]
            return pltpu.make_async_copy(src, w_v.at[b], wsem[b])

        def gather_copy(j, b):
            return pltpu.make_async_copy(
                table_ref.at[idx_v.at[b, j]],
                rows_v.at[b, pl.ds(j * _B, _B)],
                gsem[b],
            )

        def out_copy(g, b):
            return (
                pltpu.make_async_copy(
                    out_v.at[b, pl.ds(0, 8)], out_hbm.at[0, g], osem[b]
                ),
                pltpu.make_async_copy(
                    out_v.at[b, pl.ds(8, 8)], out_hbm.at[1, g], osem[b]
                ),
            )

        def combine(b):
            @plsc.parallel_loop(0, _B, unroll=2)
            def _(p):
                pcol = jnp.full((16,), p, jnp.int32)
                wv = plsc.load_gather(w_v.at[b], [iot16, pcol])
                t = [
                    rows_v[b, k * _B + p] * _bcast_lane(wv, k) for k in range(16)
                ]
                while len(t) > 1:
                    t = [t[i] + t[i + 1] for i in range(0, len(t), 2)]
                plsc.store_scatter(out_v.at[b], [iot16, pcol], t[0])

        idx_copy(base_g, 0).start()
        w_copy(base_g, 0).start()
        idx_copy(base_g + 1, 1).start()
        w_copy(base_g + 1, 1).start()
        idx_copy(base_g, 0).wait()
        for j in range(16):
            gather_copy(j, 0).start()

        @pl.loop(0, nchunks, step=2)
        def _pair(ci):
            for b in (0, 1):
                c = ci + b
                g = base_g + c
                nb = 1 - b

                @pl.when(c + 1 < nchunks)
                def _():
                    idx_copy(g + 1, nb).wait()
                    for j in range(16):
                        gather_copy(j, nb).start()

                for j in range(16):
                    gather_copy(j, b).wait()

                @pl.when(c + 2 < nchunks)
                def _():
                    idx_copy(g + 2, b).start()

                @pl.when(c >= 2)
                def _():
                    for cp in out_copy(g - 2, b):
                        cp.wait()

                w_copy(g, b).wait()
                combine(b)
                for cp in out_copy(g, b):
                    cp.start()

                @pl.when(c + 2 < nchunks)
                def _():
                    w_copy(g + 2, b).start()

        for cp in out_copy(base_g + nchunks - 2, 0):
            cp.wait()
        for cp in out_copy(base_g + nchunks - 1, 1):
            cp.wait()

    return prep, sc_relayout, sc_main


def kernel(coords, visible):
    H, W, C = visible.shape
    N = coords.shape[0]
    R = N // _B
    prep, sc_relayout, sc_main = _build(N, H, W, C)
    xs = coords[:, 0].reshape(R, _B)
    ys = coords[:, 1].reshape(R, _B)
    idx3, w3 = prep(xs, ys)
    v_tiled = (
        visible.reshape(H, W // 128, 128, C // 8, 8)
        .transpose(0, 3, 1, 4, 2)
        .reshape(H * (C // 8) * (W // 128), 8, 128)
    )
    table = sc_relayout(v_tiled).reshape(H * W, C)
    out = sc_main(table, idx3.reshape(R * 16, _B), w3.reshape(R * 16, _B))
    return out.transpose(1, 3, 0, 2).reshape(N, C)


# speedup vs baseline: 1.0611x; 1.0075x over previous
import functools

import jax
import jax.numpy as jnp
from jax import lax
from jax.experimental import pallas as pl
from jax.experimental.pallas import tpu as pltpu
from jax.experimental.pallas import tpu_sc as plsc

_NC = 2
_NS = 16
_NW = _NC * _NS
_B = 128

_DNUMS = lax.GatherDimensionNumbers(
    offset_dims=(), collapsed_slice_dims=(0,), start_index_map=(0,)
)


def _bcast_lane(v, k):
    kv = jnp.full((16, 1), k, jnp.int32)
    return lax.gather(
        v, kv, _DNUMS, (1,), mode=lax.GatherScatterMode.PROMISE_IN_BOUNDS
    )


def _hermite_weights(t):
    t2 = t * t
    t3 = t2 * t
    w0 = -0.5 * t3 + t2 - 0.5 * t
    w1 = 1.5 * t3 - 2.5 * t2 + 1.0
    w2 = -1.5 * t3 + 2.0 * t2 + 0.5 * t
    w3 = 0.5 * t3 - 0.5 * t2
    return w0, w1, w2, w3


def _prep_block(x_ref, y_ref, idx_ref, w_ref, *, Hc, Wc, bR):
    x = jnp.clip(x_ref[...], 1.0, float(Hc) - 3.0)[:, None, :]
    y = jnp.clip(y_ref[...], 1.0, float(Wc) - 3.0)[:, None, :]
    bxf = jnp.floor(x)
    byf = jnp.floor(y)
    tx = x - bxf
    ty = y - byf
    bx = bxf.astype(jnp.int32)
    by = byf.astype(jnp.int32)

    wx0, wx1, wx2, wx3 = _hermite_weights(tx)
    wy0, wy1, wy2, wy3 = _hermite_weights(ty)

    k = lax.broadcasted_iota(jnp.int32, (bR, 16, 128), 1)
    di = k // 4
    dj = k % 4

    wx = jnp.where(di == 0, wx0, jnp.where(di == 1, wx1, jnp.where(di == 2, wx2, wx3)))
    wy = jnp.where(dj == 0, wy0, jnp.where(dj == 1, wy1, jnp.where(dj == 2, wy2, wy3)))
    w_ref[...] = wx * wy
    idx_ref[...] = (bx + di - 1) * Wc + (by + dj - 1)


@functools.cache
def _build(N, H, W, C):
    R = N // _B
    bR = 256

    prep = pl.pallas_call(
        functools.partial(_prep_block, Hc=H, Wc=W, bR=bR),
        grid=(R // bR,),
        in_specs=[
            pl.BlockSpec((bR, _B), lambda i: (i, 0)),
            pl.BlockSpec((bR, _B), lambda i: (i, 0)),
        ],
        out_specs=[
            pl.BlockSpec((bR, 16, _B), lambda i: (i, 0, 0)),
            pl.BlockSpec((bR, 16, _B), lambda i: (i, 0, 0)),
        ],
        out_shape=[
            jax.ShapeDtypeStruct((R, 16, _B), jnp.int32),
            jax.ShapeDtypeStruct((R, 16, _B), jnp.float32),
        ],
    )

    nchunks = R // _NW
    mesh = plsc.VectorSubcoreMesh(core_axis_name="c", subcore_axis_name="s")
    hper = H // _NW

    @functools.partial(
        pl.kernel,
        mesh=mesh,
        out_type=jax.ShapeDtypeStruct((H * W * C // 128, 128), jnp.float32),
        scratch_types=[
            pltpu.VMEM((2, 16, 8, 128), jnp.float32),
            pltpu.VMEM((2, 128, 128), jnp.float32),
            pltpu.SemaphoreType.DMA,
            pltpu.SemaphoreType.DMA,
            pltpu.SemaphoreType.DMA,
            pltpu.SemaphoreType.DMA,
        ],
        compiler_params=pltpu.CompilerParams(
            use_tc_tiling_on_sc=False, needs_layout_passes=False
        ),
    )
    def sc_relayout(v_hbm, t_hbm, src_v, out_v, lsem0, lsem1, osem0, osem1):
        lsem = (lsem0, lsem1)
        osem = (osem0, osem1)
        iot16 = lax.iota(jnp.int32, 16)
        c8v8 = (iot16 // 8) * 8
        c8r = iot16 % 8
        wid = lax.axis_index("s") * _NC + lax.axis_index("c")
        steps = hper * 2

        def src_copies(s, b):
            h = wid * hper + s // 2
            half = s % 2
            return [
                pltpu.make_async_copy(
                    v_hbm.at[pl.ds((h * 2 + c8i) * 16 + half * 8, 8)],
                    src_v.at[b, pl.ds(c8i * 8, 8)],
                    lsem[b],
                )
                for c8i in (0, 1)
            ]

        def out_copy(s, b):
            h = wid * hper + s // 2
            half = s % 2
            r0 = pl.multiple_of(h * (W * C // 128) + half * 128, 128)
            return pltpu.make_async_copy(
                out_v.at[b], t_hbm.at[pl.ds(r0, 128)], osem[b]
            )

        for cp in src_copies(0, 0):
            cp.start()
        for cp in src_copies(1, 1):
            cp.start()

        @pl.loop(0, steps, step=2)
        def _pair(si):
            for b in (0, 1):
                s = si + b
                for cp in src_copies(s, b):
                    cp.wait()

                @pl.when(s >= 2)
                def _():
                    out_copy(s - 2, b).wait()

                @plsc.parallel_loop(0, 128, unroll=4)
                def _(r):
                    w0 = r * 8
                    idx0 = c8v8 + w0 // _B
                    for j in range(8):
                        wlv = jnp.full((16,), w0 % _B + j, jnp.int32)
                        vals = plsc.load_gather(src_v.at[b], [idx0, c8r, wlv])
                        out_v[b, r, pl.ds(j * 16, 16)] = vals

                out_copy(s, b).start()

                @pl.when(s + 2 < steps)
                def _():
                    for cp in src_copies(s + 2, b):
                        cp.start()

        out_copy(steps - 2, 0).wait()
        out_copy(steps - 1, 1).wait()

    @functools.partial(
        pl.kernel,
        mesh=mesh,
        out_type=jax.ShapeDtypeStruct((C // 8, R, 8, _B), jnp.float32),
        scratch_types=[
            pltpu.VMEM((2, 16, _B), jnp.int32),
            pltpu.VMEM((2, 16, _B), jnp.float32),
            pltpu.VMEM((2, 16 * _B, C), jnp.float32),
            pltpu.VMEM((2, C, _B), jnp.float32),
            pltpu.SemaphoreType.DMA,
            pltpu.SemaphoreType.DMA,
            pltpu.SemaphoreType.DMA,
            pltpu.SemaphoreType.DMA,
            pltpu.SemaphoreType.DMA,
            pltpu.SemaphoreType.DMA,
            pltpu.SemaphoreType.DMA,
            pltpu.SemaphoreType.DMA,
        ],
        compiler_params=pltpu.CompilerParams(
            use_tc_tiling_on_sc=False, needs_layout_passes=False
        ),
    )
    def sc_main(
        table_ref,
        idx_hbm,
        w_hbm,
        out_hbm,
        idx_v,
        w_v,
        rows_v,
        out_v,
        isem0,
        isem1,
        wsem0,
        wsem1,
        gsem0,
        gsem1,
        osem0,
        osem1,
    ):
        isem = (isem0, isem1)
        wsem = (wsem0, wsem1)
        gsem = (gsem0, gsem1)
        osem = (osem0, osem1)

        iot16 = lax.iota(jnp.int32, 16)
        wid = lax.axis_index("s") * _NC + lax.axis_index("c")
        base_g = wid * nchunks

        def idx_copy(g, b):
            src = idx_hbm.at[pl.ds(pl.multiple_of(g * 16, 16), 16)]
            return pltpu.make_async_copy(src, idx_v.at[b], isem[b])

        def w_copy(g, b):
            src = w_hbm.at[pl.ds(pl.multiple_of(g * 16, 16), 16)]
            return pltpu.make_async_copy(src, w_v.at[b], wsem[b])

        def gather_copy(j, b):
            return pltpu.make_async_copy(
                table_ref.at[idx_v.at[b, j]],
                rows_v.at[b, pl.ds(j * _B, _B)],
                gsem[b],
            )

        def out_copy(g, b):
            return (
                pltpu.make_async_copy(
                    out_v.at[b, pl.ds(0, 8)], out_hbm.at[0, g], osem[b]
                ),
                pltpu.make_async_copy(
                    out_v.at[b, pl.ds(8, 8)], out_hbm.at[1, g], osem[b]
                ),
            )

        def combine(b):
            @plsc.parallel_loop(0, _B, unroll=4)
            def _(p):
                pcol = jnp.full((16,), p, jnp.int32)
                wv = plsc.load_gather(w_v.at[b], [iot16, pcol])
                t = [
                    rows_v[b, k * _B + p] * _bcast_lane(wv, k) for k in range(16)
                ]
                while len(t) > 1:
                    t = [t[i] + t[i + 1] for i in range(0, len(t), 2)]
                plsc.store_scatter(out_v.at[b], [iot16, pcol], t[0])

        idx_copy(base_g, 0).start()
        w_copy(base_g, 0).start()
        idx_copy(base_g + 1, 1).start()
        w_copy(base_g + 1, 1).start()
        idx_copy(base_g, 0).wait()
        for j in range(16):
            gather_copy(j, 0).start()

        @pl.loop(0, nchunks, step=2)
        def _pair(ci):
            for b in (0, 1):
                c = ci + b
                g = base_g + c
                nb = 1 - b

                @pl.when(c + 1 < nchunks)
                def _():
                    idx_copy(g + 1, nb).wait()
                    for j in range(16):
                        gather_copy(j, nb).start()

                for j in range(16):
                    gather_copy(j, b).wait()

                @pl.when(c + 2 < nchunks)
                def _():
                    idx_copy(g + 2, b).start()

                @pl.when(c >= 2)
                def _():
                    for cp in out_copy(g - 2, b):
                        cp.wait()

                w_copy(g, b).wait()
                combine(b)
                for cp in out_copy(g, b):
                    cp.start()

                @pl.when(c + 2 < nchunks)
                def _():
                    w_copy(g + 2, b).start()

        for cp in out_copy(base_g + nchunks - 2, 0):
            cp.wait()
        for cp in out_copy(base_g + nchunks - 1, 1):
            cp.wait()

    return prep, sc_relayout, sc_main


def kernel(coords, visible):
    H, W, C = visible.shape
    N = coords.shape[0]
    R = N // _B
    prep, sc_relayout, sc_main = _build(N, H, W, C)
    xs = coords[:, 0].reshape(R, _B)
    ys = coords[:, 1].reshape(R, _B)
    idx3, w3 = prep(xs, ys)
    v_tiled = (
        visible.reshape(H, W // 128, 128, C // 8, 8)
        .transpose(0, 3, 1, 4, 2)
        .reshape(H * (C // 8) * (W // 128), 8, 128)
    )
    table = sc_relayout(v_tiled).reshape(H * W, C)
    out = sc_main(table, idx3.reshape(R * 16, _B), w3.reshape(R * 16, _B))
    return out.transpose(1, 3, 0, 2).reshape(N, C)
